# baseline (speedup 1.0000x reference)
import functools

import jax
import jax.numpy as jnp
from jax import lax
from jax.experimental import pallas as pl
from jax.experimental.pallas import tpu as pltpu
from jax.experimental.pallas import tpu_sc as plsc

N = 8192
K = 8192
D = 256
NB = 256
_CC_OVER_D = 0.25 / D

_PREC = lax.Precision.DEFAULT


def _argmin_body(xsq_ref, wsq_ref, x_ref, w_ref, idx_ref, loss_ref):
    m = lax.dot_general(x_ref[...], w_ref[...], (((1,), (1,)), ((), ())),
                        precision=_PREC, preferred_element_type=jnp.float32)
    d = (xsq_ref[...] + wsq_ref[...]) - 2.0 * m

    tmin = jnp.min(d, axis=1)
    gidx = lax.broadcasted_iota(jnp.int32, (NB, K), 1)
    targ = jnp.min(jnp.where(d == tmin[:, None], gidx, jnp.int32(2**31 - 1)),
                   axis=1)

    idx_ref[...] = targ[:, None]
    loss_ref[...] = tmin[:, None] * _CC_OVER_D


def _argmin_call(flat, weight, xsq, wsq):
    return pl.pallas_call(
        _argmin_body,
        grid=(N // NB,),
        in_specs=[
            pl.BlockSpec((NB, 1), lambda i: (i, 0)),
            pl.BlockSpec((1, K), lambda i: (0, 0)),
            pl.BlockSpec((NB, D), lambda i: (i, 0)),
            pl.BlockSpec((K, D), lambda i: (0, 0)),
        ],
        out_specs=[
            pl.BlockSpec((NB, 1), lambda i: (i, 0)),
            pl.BlockSpec((NB, 1), lambda i: (i, 0)),
        ],
        out_shape=[
            jax.ShapeDtypeStruct((N, 1), jnp.int32),
            jax.ShapeDtypeStruct((N, 1), jnp.float32),
        ],
    )(xsq, wsq, flat, weight)


_NC = 2
_NS = 16
_NW = _NC * _NS
_BPW = N // _NW
_CHUNK = 128
_NCH = _BPW // _CHUNK


def _sc_body(table_hbm, idx2d_hbm, out_hbm, idx_v, rows_v, sem):
    c = lax.axis_index("c")
    s = lax.axis_index("s")
    wid = s * _NC + c
    base = wid * _BPW

    pltpu.sync_copy(idx2d_hbm.at[pl.ds(wid * _NCH, _NCH)], idx_v)

    cps = [pltpu.async_copy(table_hbm.at[idx_v.at[i]],
                            rows_v.at[pl.ds(i * _CHUNK, _CHUNK)], sem)
           for i in range(_NCH)]
    for cp in cps:
        cp.wait()
    pltpu.sync_copy(rows_v, out_hbm.at[pl.ds(base, _BPW)])


@functools.cache
def _sc_gather_fn():
    return pl.kernel(
        _sc_body,
        out_type=jax.ShapeDtypeStruct((N, D), jnp.float32),
        mesh=plsc.VectorSubcoreMesh(core_axis_name="c", subcore_axis_name="s",
                                    num_cores=_NC, num_subcores=_NS),
        scratch_types=[pltpu.VMEM((_NCH, _CHUNK), jnp.int32),
                       pltpu.VMEM((_BPW, D), jnp.float32),
                       pltpu.SemaphoreType.DMA],
    )


KB2 = 1024


def _perp_body(idx_ref, out_ref, acc_ref):
    j = pl.program_id(0)
    nj = pl.num_programs(0)
    kk = lax.broadcasted_iota(jnp.int32, (1, KB2), 1) + j * KB2
    cnt = jnp.sum((idx_ref[...] == kk).astype(jnp.float32), axis=0,
                  keepdims=True)
    avg = cnt * jnp.float32(1.0 / N)
    ent = jnp.sum(avg * jnp.log(avg + 1e-10))

    @pl.when(j == 0)
    def _init():
        acc_ref[0, 0] = ent

    @pl.when(j > 0)
    def _acc():
        acc_ref[0, 0] = acc_ref[0, 0] + ent

    @pl.when(j == nj - 1)
    def _final():
        out_ref[...] = jnp.full((1, 1), jnp.exp(-acc_ref[0, 0]), jnp.float32)


def _perp_call(idx2):
    return pl.pallas_call(
        _perp_body,
        grid=(K // KB2,),
        in_specs=[pl.BlockSpec((N, 1), lambda j: (0, 0))],
        out_specs=pl.BlockSpec((1, 1), lambda j: (0, 0)),
        out_shape=jax.ShapeDtypeStruct((1, 1), jnp.float32),
        scratch_shapes=[pltpu.SMEM((1, 1), jnp.float32)],
    )(idx2)


def kernel(inputs, weight):
    x = jnp.transpose(inputs.astype(jnp.float32), (0, 2, 3, 1))
    flat = x.reshape(-1, D)
    xsq = jnp.sum(flat ** 2, axis=1, keepdims=True)
    wsq = jnp.sum(weight ** 2, axis=1).reshape(1, K)

    idx2, loss2 = _argmin_call(flat, weight, xsq, wsq)

    q = _sc_gather_fn()(weight, idx2.reshape(N // _CHUNK, _CHUNK))

    perp = _perp_call(idx2)[0, 0]

    loss = loss2.reshape(8, 32, 32)
    quantized = jnp.transpose(q.reshape(8, 32, 32, D), (0, 3, 1, 2))
    return (loss, quantized, perp, idx2)


# speedup vs baseline: 1.0749x; 1.0749x over previous
import functools

import jax
import jax.numpy as jnp
from jax import lax
from jax.experimental import pallas as pl
from jax.experimental.pallas import tpu as pltpu
from jax.experimental.pallas import tpu_sc as plsc

N = 8192
K = 8192
D = 256
NB = 512
_CC_OVER_D = 0.25 / D

_PREC = lax.Precision.DEFAULT


def _argmin_body(xsq_ref, wsq_ref, x_ref, w_ref, idx_ref, loss_ref):
    m = lax.dot_general(x_ref[...], w_ref[...], (((1,), (1,)), ((), ())),
                        precision=_PREC, preferred_element_type=jnp.float32)
    d = (xsq_ref[...] + wsq_ref[...]) - 2.0 * m

    tmin = jnp.min(d, axis=1)
    gidx = lax.broadcasted_iota(jnp.int32, (NB, K), 1)
    targ = jnp.min(jnp.where(d == tmin[:, None], gidx, jnp.int32(2**31 - 1)),
                   axis=1)

    idx_ref[...] = targ[:, None]
    loss_ref[...] = tmin[:, None] * _CC_OVER_D


def _argmin_call(flat, weight, xsq, wsq):
    return pl.pallas_call(
        _argmin_body,
        grid=(N // NB,),
        in_specs=[
            pl.BlockSpec((NB, 1), lambda i: (i, 0)),
            pl.BlockSpec((1, K), lambda i: (0, 0)),
            pl.BlockSpec((NB, D), lambda i: (i, 0)),
            pl.BlockSpec((K, D), lambda i: (0, 0)),
        ],
        out_specs=[
            pl.BlockSpec((NB, 1), lambda i: (i, 0)),
            pl.BlockSpec((NB, 1), lambda i: (i, 0)),
        ],
        out_shape=[
            jax.ShapeDtypeStruct((N, 1), jnp.int32),
            jax.ShapeDtypeStruct((N, 1), jnp.float32),
        ],
    )(xsq, wsq, flat, weight)


_NC = 2
_NS = 16
_NW = _NC * _NS
_BPW = N // _NW
_CHUNK = 128
_NCH = _BPW // _CHUNK


def _sc_body(table_hbm, idx2d_hbm, out_hbm, idx_v, rows_v, sem):
    c = lax.axis_index("c")
    s = lax.axis_index("s")
    wid = s * _NC + c
    base = wid * _BPW

    pltpu.sync_copy(idx2d_hbm.at[pl.ds(wid * _NCH, _NCH)], idx_v)

    cps = [pltpu.async_copy(table_hbm.at[idx_v.at[i]],
                            rows_v.at[pl.ds(i * _CHUNK, _CHUNK)], sem)
           for i in range(_NCH)]
    for cp in cps:
        cp.wait()
    pltpu.sync_copy(rows_v, out_hbm.at[pl.ds(base, _BPW)])


@functools.cache
def _sc_gather_fn():
    return pl.kernel(
        _sc_body,
        out_type=jax.ShapeDtypeStruct((N, D), jnp.float32),
        mesh=plsc.VectorSubcoreMesh(core_axis_name="c", subcore_axis_name="s",
                                    num_cores=_NC, num_subcores=_NS),
        scratch_types=[pltpu.VMEM((_NCH, _CHUNK), jnp.int32),
                       pltpu.VMEM((_BPW, D), jnp.float32),
                       pltpu.SemaphoreType.DMA],
    )


KB2 = 1024


def _perp_body(idx_ref, out_ref, acc_ref):
    j = pl.program_id(0)
    nj = pl.num_programs(0)
    kk = lax.broadcasted_iota(jnp.int32, (1, KB2), 1) + j * KB2
    cnt = jnp.sum((idx_ref[...] == kk).astype(jnp.float32), axis=0,
                  keepdims=True)
    avg = cnt * jnp.float32(1.0 / N)
    ent = jnp.sum(avg * jnp.log(avg + 1e-10))

    @pl.when(j == 0)
    def _init():
        acc_ref[0, 0] = ent

    @pl.when(j > 0)
    def _acc():
        acc_ref[0, 0] = acc_ref[0, 0] + ent

    @pl.when(j == nj - 1)
    def _final():
        out_ref[...] = jnp.full((1, 1), jnp.exp(-acc_ref[0, 0]), jnp.float32)


def _perp_call(idx2):
    return pl.pallas_call(
        _perp_body,
        grid=(K // KB2,),
        in_specs=[pl.BlockSpec((N, 1), lambda j: (0, 0))],
        out_specs=pl.BlockSpec((1, 1), lambda j: (0, 0)),
        out_shape=jax.ShapeDtypeStruct((1, 1), jnp.float32),
        scratch_shapes=[pltpu.SMEM((1, 1), jnp.float32)],
    )(idx2)


def kernel(inputs, weight):
    x = jnp.transpose(inputs.astype(jnp.float32), (0, 2, 3, 1))
    flat = x.reshape(-1, D)
    xsq = jnp.sum(flat ** 2, axis=1, keepdims=True)
    wsq = jnp.sum(weight ** 2, axis=1).reshape(1, K)

    idx2, loss2 = _argmin_call(flat, weight, xsq, wsq)

    q = _sc_gather_fn()(weight, idx2.reshape(N // _CHUNK, _CHUNK))

    perp = _perp_call(idx2)[0, 0]

    loss = loss2.reshape(8, 32, 32)
    quantized = jnp.transpose(q.reshape(8, 32, 32, D), (0, 3, 1, 2))
    return (loss, quantized, perp, idx2)


# speedup vs baseline: 1.0987x; 1.0221x over previous
import functools

import jax
import jax.numpy as jnp
from jax import lax
from jax.experimental import pallas as pl
from jax.experimental.pallas import tpu as pltpu
from jax.experimental.pallas import tpu_sc as plsc

N = 8192
K = 8192
D = 256
NB = 1024
_CC_OVER_D = 0.25 / D

_PREC = lax.Precision.DEFAULT


def _argmin_body(xsq_ref, wsq_ref, x_ref, w_ref, idx_ref, loss_ref):
    m = lax.dot_general(x_ref[...], w_ref[...], (((1,), (1,)), ((), ())),
                        precision=_PREC, preferred_element_type=jnp.float32)
    d = (xsq_ref[...] + wsq_ref[...]) - 2.0 * m

    tmin = jnp.min(d, axis=1)
    gidx = lax.broadcasted_iota(jnp.int32, (NB, K), 1)
    targ = jnp.min(jnp.where(d == tmin[:, None], gidx, jnp.int32(2**31 - 1)),
                   axis=1)

    idx_ref[...] = targ[:, None]
    loss_ref[...] = tmin[:, None] * _CC_OVER_D


def _argmin_call(flat, weight, xsq, wsq):
    return pl.pallas_call(
        _argmin_body,
        grid=(N // NB,),
        in_specs=[
            pl.BlockSpec((NB, 1), lambda i: (i, 0)),
            pl.BlockSpec((1, K), lambda i: (0, 0)),
            pl.BlockSpec((NB, D), lambda i: (i, 0)),
            pl.BlockSpec((K, D), lambda i: (0, 0)),
        ],
        out_specs=[
            pl.BlockSpec((NB, 1), lambda i: (i, 0)),
            pl.BlockSpec((NB, 1), lambda i: (i, 0)),
        ],
        out_shape=[
            jax.ShapeDtypeStruct((N, 1), jnp.int32),
            jax.ShapeDtypeStruct((N, 1), jnp.float32),
        ],
    )(xsq, wsq, flat, weight)


_NC = 2
_NS = 16
_NW = _NC * _NS
_BPW = N // _NW
_CHUNK = 128
_NCH = _BPW // _CHUNK


def _sc_body(table_hbm, idx2d_hbm, out_hbm, idx_v, rows_v, sem):
    c = lax.axis_index("c")
    s = lax.axis_index("s")
    wid = s * _NC + c
    base = wid * _BPW

    pltpu.sync_copy(idx2d_hbm.at[pl.ds(wid * _NCH, _NCH)], idx_v)

    cps = [pltpu.async_copy(table_hbm.at[idx_v.at[i]],
                            rows_v.at[pl.ds(i * _CHUNK, _CHUNK)], sem)
           for i in range(_NCH)]
    for cp in cps:
        cp.wait()
    pltpu.sync_copy(rows_v, out_hbm.at[pl.ds(base, _BPW)])


@functools.cache
def _sc_gather_fn():
    return pl.kernel(
        _sc_body,
        out_type=jax.ShapeDtypeStruct((N, D), jnp.float32),
        mesh=plsc.VectorSubcoreMesh(core_axis_name="c", subcore_axis_name="s",
                                    num_cores=_NC, num_subcores=_NS),
        scratch_types=[pltpu.VMEM((_NCH, _CHUNK), jnp.int32),
                       pltpu.VMEM((_BPW, D), jnp.float32),
                       pltpu.SemaphoreType.DMA],
    )


KB2 = 1024


def _perp_body(idx_ref, out_ref, acc_ref):
    j = pl.program_id(0)
    nj = pl.num_programs(0)
    kk = lax.broadcasted_iota(jnp.int32, (1, KB2), 1) + j * KB2
    cnt = jnp.sum((idx_ref[...] == kk).astype(jnp.float32), axis=0,
                  keepdims=True)
    avg = cnt * jnp.float32(1.0 / N)
    ent = jnp.sum(avg * jnp.log(avg + 1e-10))

    @pl.when(j == 0)
    def _init():
        acc_ref[0, 0] = ent

    @pl.when(j > 0)
    def _acc():
        acc_ref[0, 0] = acc_ref[0, 0] + ent

    @pl.when(j == nj - 1)
    def _final():
        out_ref[...] = jnp.full((1, 1), jnp.exp(-acc_ref[0, 0]), jnp.float32)


def _perp_call(idx2):
    return pl.pallas_call(
        _perp_body,
        grid=(K // KB2,),
        in_specs=[pl.BlockSpec((N, 1), lambda j: (0, 0))],
        out_specs=pl.BlockSpec((1, 1), lambda j: (0, 0)),
        out_shape=jax.ShapeDtypeStruct((1, 1), jnp.float32),
        scratch_shapes=[pltpu.SMEM((1, 1), jnp.float32)],
    )(idx2)


def kernel(inputs, weight):
    x = jnp.transpose(inputs.astype(jnp.float32), (0, 2, 3, 1))
    flat = x.reshape(-1, D)
    xsq = jnp.sum(flat ** 2, axis=1, keepdims=True)
    wsq = jnp.sum(weight ** 2, axis=1).reshape(1, K)

    idx2, loss2 = _argmin_call(flat, weight, xsq, wsq)

    q = _sc_gather_fn()(weight, idx2.reshape(N // _CHUNK, _CHUNK))

    perp = _perp_call(idx2)[0, 0]

    loss = loss2.reshape(8, 32, 32)
    quantized = jnp.transpose(q.reshape(8, 32, 32, D), (0, 3, 1, 2))
    return (loss, quantized, perp, idx2)
